# baseline (speedup 1.0000x reference)
import functools

import jax
import jax.numpy as jnp
from jax import lax
from jax.experimental import pallas as pl
from jax.experimental.pallas import tpu as pltpu
from jax.experimental.pallas import tpu_sc as plsc

K = 16
ALPHA = 0.1
N = 10000
E = 320000
D = 128

NC = 2
NS = 16
NP = 10240
RPT = NP // NS

EPW = E // (NC * NS)
CH = 80
NCHUNK = EPW // CH

PCH = 64
PEPW = 10240
EPAD = PEPW * NC * NS
PCPT = PEPW // PCH
PG = 20
PNG = PCPT // PG

_MESH = plsc.VectorSubcoreMesh(core_axis_name="c", subcore_axis_name="s")


@functools.partial(
    pl.kernel,
    out_type=jax.ShapeDtypeStruct((NC, NP), jnp.float32),
    mesh=_MESH,
    scratch_types=[
        pltpu.VMEM((NCHUNK, CH), jnp.int32),
        pltpu.VMEM((CH,), jnp.float32),
        pltpu.VMEM((RPT,), jnp.float32),
        pltpu.VMEM_SHARED((NP,), jnp.float32),
    ],
)
def _degree_sc(col_hbm, out_hbm, colv, ones_v, zeros_v, acc):
    cid = lax.axis_index("c")
    sid = lax.axis_index("s")

    one16 = jnp.ones((16,), jnp.float32)
    zero16 = jnp.zeros((16,), jnp.float32)
    for i in range(CH // 16):
        ones_v[pl.ds(i * 16, 16)] = one16

    def _zero(i, _):
        zeros_v[pl.ds(i * 16, 16)] = zero16
        return 0

    lax.fori_loop(0, RPT // 16, _zero, 0)
    pltpu.sync_copy(zeros_v, acc.at[pl.ds(sid * RPT, RPT)])
    plsc.subcore_barrier()

    pltpu.sync_copy(col_hbm.at[cid, sid], colv)

    def _chunk(i, _):
        pltpu.sync_copy(ones_v, acc.at[colv.at[i]], add=True)
        return 0

    lax.fori_loop(0, NCHUNK, _chunk, 0)
    plsc.subcore_barrier()
    pltpu.sync_copy(acc.at[pl.ds(sid * RPT, RPT)],
                    out_hbm.at[cid, pl.ds(sid * RPT, RPT)])


@functools.partial(
    pl.kernel,
    out_type=jax.ShapeDtypeStruct((NC, NP, D), jnp.float32),
    mesh=_MESH,
    scratch_types=[
        pltpu.VMEM((PG, PCH), jnp.int32),
        pltpu.VMEM((PG, PCH), jnp.int32),
        pltpu.VMEM((4, PCH, D), jnp.float32),
        pltpu.VMEM_SHARED((NP, D), jnp.float32),
        pltpu.SemaphoreType.DMA,
        pltpu.SemaphoreType.DMA,
    ],
)
def _propagate_sc(g_hbm, b1_hbm, row_hbm, col_hbm, out_hbm,
                  rowv, colv, bufs, acc, semg, sems):
    cid = lax.axis_index("c")
    sid = lax.axis_index("s")

    rows = pl.ds(sid * RPT, RPT)

    @pl.when(cid == 0)
    def _():
        pltpu.sync_copy(g_hbm.at[rows], acc.at[rows])

    @pl.when(cid == 1)
    def _():
        pltpu.sync_copy(b1_hbm.at[rows], acc.at[rows])

    plsc.subcore_barrier()

    def _wait_gather(i, p):
        pltpu.make_async_copy(g_hbm.at[rowv.at[i]], bufs.at[p], semg).wait()

    def _scatter(i, p):
        pltpu.async_copy(bufs.at[p], acc.at[colv.at[i]], sems, add=True)

    def _wait_scatter():
        pltpu.make_async_copy(bufs.at[0], acc.at[colv.at[0]], sems).wait()

    def _group(g, _):
        pltpu.sync_copy(row_hbm.at[cid, sid, g], rowv)
        pltpu.sync_copy(col_hbm.at[cid, sid, g], colv)
        pltpu.async_copy(g_hbm.at[rowv.at[0]], bufs.at[0], semg)
        pltpu.async_copy(g_hbm.at[rowv.at[1]], bufs.at[1], semg)
        pltpu.async_copy(g_hbm.at[rowv.at[2]], bufs.at[2], semg)

        def _step(i, _):
            p = lax.rem(i, 4)
            _wait_gather(i, p)
            _scatter(i, p)

            @pl.when(i >= 1)
            def _():
                _wait_scatter()

            pltpu.async_copy(g_hbm.at[rowv.at[i + 3]],
                             bufs.at[lax.rem(i + 3, 4)], semg)
            return 0

        lax.fori_loop(0, PG - 3, _step, 0)
        for t in (PG - 3, PG - 2, PG - 1):
            _wait_gather(t, t % 4)
            _scatter(t, t % 4)
            _wait_scatter()
        _wait_scatter()
        return 0

    lax.fori_loop(0, PNG, _group, 0)

    plsc.subcore_barrier()
    pltpu.sync_copy(acc.at[rows], out_hbm.at[cid, rows])


def _prep_body(d0_ref, d1_ref, xp_ref, g0_ref, b1_ref, wv_ref, wf_ref):
    deg = d0_ref[...] + d1_ref[...] + 1.0
    dis = lax.rsqrt(deg)
    g0_ref[...] = dis * xp_ref[...]
    b1_ref[...] = (ALPHA / (1.0 - ALPHA)) * jnp.sqrt(deg) * xp_ref[...]
    wv_ref[...] = (1.0 - ALPHA) / deg
    wf_ref[...] = (1.0 - ALPHA) * dis


def _combine_body(p0_ref, p1_ref, scale_ref, o_ref):
    o_ref[...] = scale_ref[...] * (p0_ref[...] + p1_ref[...])


_BR = 2048
_GRID = NP // _BR


def _rows_spec(width):
    return pl.BlockSpec((_BR, width), lambda i: (i, 0))


_prep_tc = pl.pallas_call(
    _prep_body,
    grid=(_GRID,),
    in_specs=[_rows_spec(1), _rows_spec(1), _rows_spec(D)],
    out_specs=[_rows_spec(D), _rows_spec(D), _rows_spec(1), _rows_spec(1)],
    out_shape=[
        jax.ShapeDtypeStruct((NP, D), jnp.float32),
        jax.ShapeDtypeStruct((NP, D), jnp.float32),
        jax.ShapeDtypeStruct((NP, 1), jnp.float32),
        jax.ShapeDtypeStruct((NP, 1), jnp.float32),
    ],
)

_combine_tc = pl.pallas_call(
    _combine_body,
    grid=(_GRID,),
    in_specs=[_rows_spec(D), _rows_spec(D), _rows_spec(1)],
    out_specs=_rows_spec(D),
    out_shape=jax.ShapeDtypeStruct((NP, D), jnp.float32),
)


def kernel(x, edge_index):
    pad_n = EPAD - E
    pad_edges = jnp.stack([jnp.zeros((pad_n,), edge_index.dtype),
                           jnp.full((pad_n,), NP - 1, edge_index.dtype)])
    eip = jnp.concatenate([edge_index, pad_edges], axis=1)
    row = eip[0].reshape(NC, NS, PNG, PG, PCH)
    col = eip[1].reshape(NC, NS, PNG, PG, PCH)
    col4 = edge_index[1].reshape(NC, NS, NCHUNK, CH)
    xp = jnp.pad(x, ((0, NP - N), (0, 0)))

    degp = _degree_sc(col4)
    g0, b1, wv, wf = _prep_tc(degp[0][:, None], degp[1][:, None], xp)

    g = g0
    for k in range(K):
        parts = _propagate_sc(g, b1, row, col)
        g = _combine_tc(parts[0], parts[1], wv if k < K - 1 else wf)
    return g[:N]


# speedup vs baseline: 4.2747x; 4.2747x over previous
import functools

import jax
import jax.numpy as jnp
from jax import lax
from jax.experimental import pallas as pl
from jax.experimental.pallas import tpu as pltpu
from jax.experimental.pallas import tpu_sc as plsc

K = 16
ALPHA = 0.1
N = 10000
E = 320000
D = 128

NC = 2
NS = 16
NP = 10240
RPT = NP // NS

EPW = E // (NC * NS)
CH = 80
NCHUNK = EPW // CH
IG = 5
NIG = NCHUNK // IG

DCH = 128
DCPT = 80
DEPW = DCH * DCPT
DEPAD = DEPW * NC * NS

_MESH = plsc.VectorSubcoreMesh(core_axis_name="c", subcore_axis_name="s")


@functools.partial(
    pl.kernel,
    out_type=jax.ShapeDtypeStruct((NC, NP), jnp.float32),
    mesh=_MESH,
    scratch_types=[
        pltpu.VMEM((DCPT, DCH), jnp.int32),
        pltpu.VMEM((DCH,), jnp.float32),
        pltpu.VMEM((RPT,), jnp.float32),
        pltpu.VMEM_SHARED((NP,), jnp.float32),
        pltpu.SemaphoreType.DMA,
    ],
)
def _degree_sc(col_hbm, out_hbm, colv, ones_v, zeros_v, acc, sem):
    cid = lax.axis_index("c")
    sid = lax.axis_index("s")

    one16 = jnp.ones((16,), jnp.float32)
    zero16 = jnp.zeros((16,), jnp.float32)
    for i in range(DCH // 16):
        ones_v[pl.ds(i * 16, 16)] = one16

    def _zero(i, _):
        zeros_v[pl.ds(i * 16, 16)] = zero16
        return 0

    lax.fori_loop(0, RPT // 16, _zero, 0)
    pltpu.sync_copy(zeros_v, acc.at[pl.ds(sid * RPT, RPT)])
    pltpu.sync_copy(col_hbm.at[cid, sid], colv)
    plsc.subcore_barrier()

    def _chunk(i, _):
        pltpu.async_copy(ones_v, acc.at[colv.at[i]], sem, add=True)

        @pl.when(i >= 16)
        def _():
            pltpu.make_async_copy(ones_v, acc.at[colv.at[0]], sem).wait()

        return 0

    lax.fori_loop(0, DCPT, _chunk, 0)

    def _drain(i, _):
        pltpu.make_async_copy(ones_v, acc.at[colv.at[0]], sem).wait()
        return 0

    lax.fori_loop(0, 16, _drain, 0)
    plsc.subcore_barrier()
    pltpu.sync_copy(acc.at[pl.ds(sid * RPT, RPT)],
                    out_hbm.at[cid, pl.ds(sid * RPT, RPT)])


@functools.partial(
    pl.kernel,
    out_type=jax.ShapeDtypeStruct((NC, NP, D), jnp.float32),
    mesh=_MESH,
    scratch_types=[
        pltpu.VMEM((3, IG, CH), jnp.int32),
        pltpu.VMEM((3, IG, CH), jnp.int32),
        pltpu.VMEM((3, CH, D), jnp.float32),
        pltpu.VMEM_SHARED((NP, D), jnp.float32),
        pltpu.SemaphoreType.DMA,
        pltpu.SemaphoreType.DMA,
        pltpu.SemaphoreType.DMA,
    ],
)
def _propagate_sc(g_hbm, b1_hbm, row_hbm, col_hbm, out_hbm,
                  rowv, colv, bufs, acc, semg, sems, semi):
    cid = lax.axis_index("c")
    sid = lax.axis_index("s")
    rows = pl.ds(sid * RPT, RPT)

    def _gather(i, islot, bslot):
        pltpu.async_copy(g_hbm.at[rowv.at[islot, i]], bufs.at[bslot], semg)

    def _wait_gather():
        pltpu.make_async_copy(g_hbm.at[rowv.at[0, 0]], bufs.at[0], semg).wait()

    def _scatter(i, islot, bslot):
        pltpu.async_copy(bufs.at[bslot], acc.at[colv.at[islot, i]], sems,
                         add=True)

    def _wait_scatter():
        pltpu.make_async_copy(bufs.at[0], acc.at[colv.at[0, 0]], sems).wait()

    def _fetch_idx(grp, slot, sem):
        pltpu.async_copy(row_hbm.at[cid, sid, grp], rowv.at[slot], sem)
        pltpu.async_copy(col_hbm.at[cid, sid, grp], colv.at[slot], sem)

    def _wait_idx():
        pltpu.make_async_copy(row_hbm.at[cid, sid, 0], rowv.at[0], semi).wait()
        pltpu.make_async_copy(col_hbm.at[cid, sid, 0], colv.at[0], semi).wait()

    pltpu.sync_copy(row_hbm.at[cid, sid, 0], rowv.at[0])
    pltpu.sync_copy(col_hbm.at[cid, sid, 0], colv.at[0])
    _gather(0, 0, 0)
    _gather(1, 0, 1)
    _fetch_idx(1, 1, semi)

    @pl.when(cid == 0)
    def _():
        pltpu.sync_copy(g_hbm.at[rows], acc.at[rows])

    @pl.when(cid == 1)
    def _():
        pltpu.sync_copy(b1_hbm.at[rows], acc.at[rows])

    plsc.subcore_barrier()

    def _step(i, _):
        q = lax.div(i, IG)
        r = lax.rem(i, IG)
        _wait_gather()
        _scatter(r, lax.rem(q, 3), lax.rem(i, 3))

        @pl.when(i >= 1)
        def _():
            _wait_scatter()

        @pl.when(r == IG - 2)
        def _():
            @pl.when(i <= NCHUNK - 7)
            def _():
                _wait_idx()

            @pl.when(i <= NCHUNK - 12)
            def _():
                _fetch_idx(q + 2, lax.rem(q + 2, 3), semi)

        j = i + 2
        _gather(lax.rem(j, IG), lax.rem(lax.div(j, IG), 3), lax.rem(j, 3))
        return 0

    lax.fori_loop(0, NCHUNK - 2, _step, 0)
    qs = (NIG - 1) % 3
    _wait_gather()
    _scatter(IG - 2, qs, (NCHUNK - 2) % 3)
    _wait_scatter()
    _wait_gather()
    _scatter(IG - 1, qs, (NCHUNK - 1) % 3)
    _wait_scatter()
    _wait_scatter()

    plsc.subcore_barrier()
    pltpu.sync_copy(acc.at[rows], out_hbm.at[cid, rows])


def _prep_body(d0_ref, d1_ref, xp_ref, g0_ref, b1_ref, wv_ref, wf_ref):
    deg = d0_ref[...] + d1_ref[...] + 1.0
    dis = lax.rsqrt(deg)
    g0_ref[...] = dis * xp_ref[...]
    b1_ref[...] = (ALPHA / (1.0 - ALPHA)) * jnp.sqrt(deg) * xp_ref[...]
    wv_ref[...] = (1.0 - ALPHA) / deg
    wf_ref[...] = (1.0 - ALPHA) * dis


def _combine_body(p0_ref, p1_ref, scale_ref, o_ref):
    o_ref[...] = scale_ref[...] * (p0_ref[...] + p1_ref[...])


_BR = 2048
_BRF = 2000


def _rows_spec(width, rows=_BR):
    return pl.BlockSpec((rows, width), lambda i: (i, 0))


_prep_tc = pl.pallas_call(
    _prep_body,
    grid=(NP // _BR,),
    in_specs=[_rows_spec(1), _rows_spec(1), _rows_spec(D)],
    out_specs=[_rows_spec(D), _rows_spec(D), _rows_spec(1), _rows_spec(1)],
    out_shape=[
        jax.ShapeDtypeStruct((NP, D), jnp.float32),
        jax.ShapeDtypeStruct((NP, D), jnp.float32),
        jax.ShapeDtypeStruct((NP, 1), jnp.float32),
        jax.ShapeDtypeStruct((NP, 1), jnp.float32),
    ],
)

_combine_tc = pl.pallas_call(
    _combine_body,
    grid=(NP // _BR,),
    in_specs=[_rows_spec(D), _rows_spec(D), _rows_spec(1)],
    out_specs=_rows_spec(D),
    out_shape=jax.ShapeDtypeStruct((NP, D), jnp.float32),
)

_final_tc = pl.pallas_call(
    _combine_body,
    grid=(N // _BRF,),
    in_specs=[_rows_spec(D, _BRF), _rows_spec(D, _BRF), _rows_spec(1, _BRF)],
    out_specs=_rows_spec(D, _BRF),
    out_shape=jax.ShapeDtypeStruct((N, D), jnp.float32),
)


def kernel(x, edge_index):
    row = edge_index[0].reshape(NC, NS, NIG, IG, CH)
    col = edge_index[1].reshape(NC, NS, NIG, IG, CH)
    cold = jnp.concatenate(
        [edge_index[1],
         jnp.full((DEPAD - E,), NP - 1, edge_index.dtype)]
    ).reshape(NC, NS, DCPT, DCH)
    xp = jnp.pad(x, ((0, NP - N), (0, 0)))

    degp = _degree_sc(cold)
    g0, b1, wv, wf = _prep_tc(degp[0][:, None], degp[1][:, None], xp)

    g = g0
    for k in range(K - 1):
        parts = _propagate_sc(g, b1, row, col)
        g = _combine_tc(parts[0], parts[1], wv)
    parts = _propagate_sc(g, b1, row, col)
    return _final_tc(parts[0], parts[1], wf)
